# Initial kernel scaffold; baseline (speedup 1.0000x reference)
import jax
import jax.numpy as jnp
from jax.experimental import pallas as pl


def kernel(hidden_states, Wq, Wk, Wv, Wo, key_horizons, query_horizons):
    raise NotImplementedError("write your pallas kernel here")


# speedup vs baseline: 1.2576x; 1.2576x over previous
import functools
import math

import jax
import jax.numpy as jnp
from jax import lax
from jax.experimental import pallas as pl
from jax.experimental.pallas import tpu as pltpu
from jax.experimental.pallas import tpu_sc as plsc

_N_HEADS = 16
_HEAD_DIM = 128
_N_BUCKETS = 256
_N_REP = 4
_DR = _N_REP * _HEAD_DIM


def _mm_nt_body(a_ref, b_ref, o_ref):
    o_ref[...] = lax.dot_general(
        a_ref[...], b_ref[...], (((1,), (1,)), ((), ())),
        preferred_element_type=jnp.float32)


def _matmul_nt(a, b, block_n=512):
    m, k = a.shape
    n, k2 = b.shape
    grid = (n // block_n,)
    return pl.pallas_call(
        _mm_nt_body,
        grid=grid,
        in_specs=[
            pl.BlockSpec((m, k), lambda j: (0, 0)),
            pl.BlockSpec((block_n, k), lambda j: (j, 0)),
        ],
        out_specs=pl.BlockSpec((m, block_n), lambda j: (0, j)),
        out_shape=jax.ShapeDtypeStruct((m, n), jnp.float32),
    )(a, b)


def _attn_body(q_ref, k_ref, v_ref, o_ref, *, scale):
    q = q_ref[0]
    k = k_ref[0]
    s = lax.dot_general(q, k, (((1,), (1,)), ((), ())),
                        preferred_element_type=jnp.float32) * scale
    m = jnp.max(s, axis=1, keepdims=True)
    p = jnp.exp(s - m)
    l = jnp.sum(p, axis=1, keepdims=True)
    o = lax.dot_general(p, v_ref[0], (((1,), (0,)), ((), ())),
                        preferred_element_type=jnp.float32)
    o_ref[0] = o / l


def _attention(q, k, v, block_q=256):
    h, s, d = q.shape
    grid = (h, s // block_q)
    return pl.pallas_call(
        functools.partial(_attn_body, scale=1.0 / math.sqrt(d)),
        grid=grid,
        in_specs=[
            pl.BlockSpec((1, block_q, d), lambda hh, qi: (hh, qi, 0)),
            pl.BlockSpec((1, s, d), lambda hh, qi: (hh, 0, 0)),
            pl.BlockSpec((1, s, d), lambda hh, qi: (hh, 0, 0)),
        ],
        out_specs=pl.BlockSpec((1, block_q, d), lambda hh, qi: (hh, qi, 0)),
        out_shape=jax.ShapeDtypeStruct((h, s, d), jnp.float32),
    )(q, k, v)


def _sc_bucket(v_r, kh, qh):
    s_len = v_r.shape[0]
    n_tiles = 16
    rows_pt = s_len // n_tiles
    bk_pt = _N_BUCKETS // n_tiles
    sl = _DR // 128

    v3 = v_r.reshape(s_len, sl, 128)
    kh2 = kh.reshape(n_tiles, rows_pt)
    qh2 = qh.reshape(n_tiles, rows_pt)
    ones3 = jnp.ones((rows_pt, 1, 128), jnp.float32)

    mesh = plsc.VectorSubcoreMesh(core_axis_name="c", subcore_axis_name="s")

    @functools.partial(
        pl.kernel,
        out_type=(
            jax.ShapeDtypeStruct((s_len, sl, 128), jnp.float32),
            jax.ShapeDtypeStruct((_N_BUCKETS, sl, 128), jnp.float32),
        ),
        mesh=mesh,
        scratch_types=[
            pltpu.VMEM((rows_pt, sl, 128), jnp.float32),
            pltpu.VMEM((1, rows_pt), jnp.int32),
            pltpu.VMEM((1, rows_pt), jnp.int32),
            pltpu.VMEM((rows_pt, 1, 128), jnp.float32),
            pltpu.VMEM((bk_pt, sl, 128), jnp.float32),
            pltpu.VMEM((bk_pt, 1, 128), jnp.float32),
            pltpu.VMEM_SHARED((_N_BUCKETS, sl, 128), jnp.float32),
            pltpu.VMEM_SHARED((_N_BUCKETS, 1, 128), jnp.float32),
            pltpu.SemaphoreType.DMA,
        ],
    )
    def bucket_kernel(v_hbm, kh_hbm, qh_hbm, ones_hbm, out_hbm, pooled_hbm,
                      v_tile, kh_v, qh_v, ones_v, pool_t, cnt_t,
                      acc, acc_cnt, sem):
        cid = lax.axis_index("c")
        sid = lax.axis_index("s")

        @pl.when(cid == 0)
        def _():
            base = sid * rows_pt
            b0 = sid * bk_pt
            zero = jnp.zeros((16,), jnp.float32)

            for r in range(bk_pt):
                for c in range(0, 128, 16):
                    cnt_t[r, 0, pl.ds(c, 16)] = zero
                    for s in range(sl):
                        pool_t[r, s, pl.ds(c, 16)] = zero
            pltpu.sync_copy(pool_t, acc.at[pl.ds(b0, bk_pt)])
            pltpu.sync_copy(cnt_t, acc_cnt.at[pl.ds(b0, bk_pt)])

            pltpu.sync_copy(v_hbm.at[pl.ds(base, rows_pt)], v_tile)
            pltpu.sync_copy(kh_hbm.at[pl.ds(sid, 1)], kh_v)
            pltpu.sync_copy(qh_hbm.at[pl.ds(sid, 1)], qh_v)
            pltpu.sync_copy(ones_hbm, ones_v)

            plsc.subcore_barrier()

            pltpu.sync_copy(v_tile, acc.at[kh_v.at[0]], add=True)
            pltpu.sync_copy(ones_v, acc_cnt.at[kh_v.at[0]], add=True)

            plsc.subcore_barrier()

            pltpu.sync_copy(acc.at[pl.ds(b0, bk_pt)], pool_t)
            pltpu.sync_copy(acc_cnt.at[pl.ds(b0, bk_pt)], cnt_t)
            for r in range(bk_pt):
                rec = 1.0 / jnp.maximum(cnt_t[r, 0, pl.ds(0, 16)], 1.0)
                for s in range(sl):
                    for c in range(0, 128, 16):
                        pool_t[r, s, pl.ds(c, 16)] = (
                            pool_t[r, s, pl.ds(c, 16)] * rec)
            pltpu.sync_copy(pool_t, pooled_hbm.at[pl.ds(b0, bk_pt)])

            plsc.subcore_barrier()

            pltpu.async_copy(pooled_hbm.at[qh_v.at[0]], v_tile, sem).wait()
            pltpu.sync_copy(v_tile, out_hbm.at[pl.ds(base, rows_pt)])

    return bucket_kernel(v3, kh2, qh2, ones3)[0].reshape(s_len, _DR)


def kernel(hidden_states, Wq, Wk, Wv, Wo, key_horizons, query_horizons):
    b, s_len, h_dim = hidden_states.shape
    d = _HEAD_DIM
    n_keep = _N_HEADS - _N_REP
    x = hidden_states.reshape(s_len, h_dim)

    w_cat = jnp.concatenate([Wq[_DR:], Wk[_DR:], Wv], axis=0)
    qkv = _matmul_nt(x, w_cat)

    q12 = qkv[:, : n_keep * d]
    k12 = qkv[:, n_keep * d: 2 * n_keep * d]
    v_all = qkv[:, 2 * n_keep * d:]
    v_r = v_all[:, :_DR]

    qh3 = q12.reshape(s_len, n_keep, d).transpose(1, 0, 2)
    kh3 = k12.reshape(s_len, n_keep, d).transpose(1, 0, 2)
    vh3 = v_all[:, _DR:].reshape(s_len, n_keep, d).transpose(1, 0, 2)

    bucket_out = _sc_bucket(v_r, key_horizons, query_horizons)
    attn = _attention(qh3, kh3, vh3)
    attn_flat = attn.transpose(1, 0, 2).reshape(s_len, n_keep * d)

    cat = jnp.concatenate([bucket_out, attn_flat], axis=1)
    out = _matmul_nt(cat, Wo)
    return out.reshape(b, s_len, h_dim)


# speedup vs baseline: 1.4205x; 1.1296x over previous
import functools
import math

import jax
import jax.numpy as jnp
from jax import lax
from jax.experimental import pallas as pl
from jax.experimental.pallas import tpu as pltpu
from jax.experimental.pallas import tpu_sc as plsc

_N_HEADS = 16
_HEAD_DIM = 128
_N_BUCKETS = 256
_N_REP = 4
_DR = _N_REP * _HEAD_DIM


def _mm_nt_body(a_ref, b_ref, o_ref):
    o_ref[...] = lax.dot_general(
        a_ref[...], b_ref[...], (((1,), (1,)), ((), ())),
        preferred_element_type=jnp.float32)


def _matmul_nt(a, b, block_n=512):
    m, k = a.shape
    n, k2 = b.shape
    a = a.astype(jnp.bfloat16)
    b = b.astype(jnp.bfloat16)
    grid = (n // block_n,)
    return pl.pallas_call(
        _mm_nt_body,
        grid=grid,
        in_specs=[
            pl.BlockSpec((m, k), lambda j: (0, 0)),
            pl.BlockSpec((block_n, k), lambda j: (j, 0)),
        ],
        out_specs=pl.BlockSpec((m, block_n), lambda j: (0, j)),
        out_shape=jax.ShapeDtypeStruct((m, n), jnp.float32),
    )(a, b)


def _attn_body(q_ref, k_ref, v_ref, o_ref, *, scale):
    q = q_ref[0]
    k = k_ref[0]
    s = lax.dot_general(q, k, (((1,), (1,)), ((), ())),
                        preferred_element_type=jnp.float32) * scale
    m = jnp.max(s, axis=1, keepdims=True)
    p = jnp.exp(s - m)
    l = jnp.sum(p, axis=1, keepdims=True)
    o = lax.dot_general(p.astype(jnp.bfloat16), v_ref[0],
                        (((1,), (0,)), ((), ())),
                        preferred_element_type=jnp.float32)
    o_ref[0] = o / l


def _attention(q, k, v, block_q=256):
    h, s, d = q.shape
    q = q.astype(jnp.bfloat16)
    k = k.astype(jnp.bfloat16)
    v = v.astype(jnp.bfloat16)
    grid = (h, s // block_q)
    return pl.pallas_call(
        functools.partial(_attn_body, scale=1.0 / math.sqrt(d)),
        grid=grid,
        in_specs=[
            pl.BlockSpec((1, block_q, d), lambda hh, qi: (hh, qi, 0)),
            pl.BlockSpec((1, s, d), lambda hh, qi: (hh, 0, 0)),
            pl.BlockSpec((1, s, d), lambda hh, qi: (hh, 0, 0)),
        ],
        out_specs=pl.BlockSpec((1, block_q, d), lambda hh, qi: (hh, qi, 0)),
        out_shape=jax.ShapeDtypeStruct((h, s, d), jnp.float32),
    )(q, k, v)


def _sc_bucket(v_r, kh, qh):
    s_len = v_r.shape[0]
    n_tiles = 16
    rows_pt = s_len // n_tiles
    bk_pt = _N_BUCKETS // n_tiles
    sl = _DR // 128

    v3 = v_r.reshape(s_len, sl, 128)
    kh2 = kh.reshape(n_tiles, rows_pt)
    qh2 = qh.reshape(n_tiles, rows_pt)
    ones3 = jnp.ones((rows_pt, 1, 128), jnp.float32)

    mesh = plsc.VectorSubcoreMesh(core_axis_name="c", subcore_axis_name="s")

    @functools.partial(
        pl.kernel,
        out_type=(
            jax.ShapeDtypeStruct((s_len, sl, 128), jnp.float32),
            jax.ShapeDtypeStruct((_N_BUCKETS, sl, 128), jnp.float32),
        ),
        mesh=mesh,
        scratch_types=[
            pltpu.VMEM((rows_pt, sl, 128), jnp.float32),
            pltpu.VMEM((1, rows_pt), jnp.int32),
            pltpu.VMEM((1, rows_pt), jnp.int32),
            pltpu.VMEM((rows_pt, 1, 128), jnp.float32),
            pltpu.VMEM((bk_pt, sl, 128), jnp.float32),
            pltpu.VMEM((bk_pt, 1, 128), jnp.float32),
            pltpu.VMEM_SHARED((_N_BUCKETS, sl, 128), jnp.float32),
            pltpu.VMEM_SHARED((_N_BUCKETS, 1, 128), jnp.float32),
            pltpu.SemaphoreType.DMA,
        ],
    )
    def bucket_kernel(v_hbm, kh_hbm, qh_hbm, ones_hbm, out_hbm, pooled_hbm,
                      v_tile, kh_v, qh_v, ones_v, pool_t, cnt_t,
                      acc, acc_cnt, sem):
        cid = lax.axis_index("c")
        sid = lax.axis_index("s")

        @pl.when(cid == 0)
        def _():
            base = sid * rows_pt
            b0 = sid * bk_pt
            zero = jnp.zeros((16,), jnp.float32)

            for r in range(bk_pt):
                for c in range(0, 128, 16):
                    cnt_t[r, 0, pl.ds(c, 16)] = zero
                    for s in range(sl):
                        pool_t[r, s, pl.ds(c, 16)] = zero
            pltpu.sync_copy(pool_t, acc.at[pl.ds(b0, bk_pt)])
            pltpu.sync_copy(cnt_t, acc_cnt.at[pl.ds(b0, bk_pt)])

            pltpu.sync_copy(v_hbm.at[pl.ds(base, rows_pt)], v_tile)
            pltpu.sync_copy(kh_hbm.at[pl.ds(sid, 1)], kh_v)
            pltpu.sync_copy(qh_hbm.at[pl.ds(sid, 1)], qh_v)
            pltpu.sync_copy(ones_hbm, ones_v)

            plsc.subcore_barrier()

            pltpu.sync_copy(v_tile, acc.at[kh_v.at[0]], add=True)
            pltpu.sync_copy(ones_v, acc_cnt.at[kh_v.at[0]], add=True)

            plsc.subcore_barrier()

            pltpu.sync_copy(acc.at[pl.ds(b0, bk_pt)], pool_t)
            pltpu.sync_copy(acc_cnt.at[pl.ds(b0, bk_pt)], cnt_t)
            for r in range(bk_pt):
                rec = 1.0 / jnp.maximum(cnt_t[r, 0, pl.ds(0, 16)], 1.0)
                for s in range(sl):
                    for c in range(0, 128, 16):
                        pool_t[r, s, pl.ds(c, 16)] = (
                            pool_t[r, s, pl.ds(c, 16)] * rec)
            pltpu.sync_copy(pool_t, pooled_hbm.at[pl.ds(b0, bk_pt)])

            plsc.subcore_barrier()

            pltpu.async_copy(pooled_hbm.at[qh_v.at[0]], v_tile, sem).wait()
            pltpu.sync_copy(v_tile, out_hbm.at[pl.ds(base, rows_pt)])

    return bucket_kernel(v3, kh2, qh2, ones3)[0].reshape(s_len, _DR)


def kernel(hidden_states, Wq, Wk, Wv, Wo, key_horizons, query_horizons):
    b, s_len, h_dim = hidden_states.shape
    d = _HEAD_DIM
    n_keep = _N_HEADS - _N_REP
    x = hidden_states.reshape(s_len, h_dim)

    w_cat = jnp.concatenate([Wq[_DR:], Wk[_DR:], Wv], axis=0)
    qkv = _matmul_nt(x, w_cat)

    q12 = qkv[:, : n_keep * d]
    k12 = qkv[:, n_keep * d: 2 * n_keep * d]
    v_all = qkv[:, 2 * n_keep * d:]
    v_r = v_all[:, :_DR]

    qh3 = q12.reshape(s_len, n_keep, d).transpose(1, 0, 2)
    kh3 = k12.reshape(s_len, n_keep, d).transpose(1, 0, 2)
    vh3 = v_all[:, _DR:].reshape(s_len, n_keep, d).transpose(1, 0, 2)

    bucket_out = _sc_bucket(v_r, key_horizons, query_horizons)
    attn = _attention(qh3, kh3, vh3)
    attn_flat = attn.transpose(1, 0, 2).reshape(s_len, n_keep * d)

    cat = jnp.concatenate([bucket_out, attn_flat], axis=1)
    out = _matmul_nt(cat, Wo)
    return out.reshape(b, s_len, h_dim)


# speedup vs baseline: 2.0869x; 1.4691x over previous
import functools
import math

import jax
import jax.numpy as jnp
from jax import lax
from jax.experimental import pallas as pl
from jax.experimental.pallas import tpu as pltpu
from jax.experimental.pallas import tpu_sc as plsc

_N_HEADS = 16
_HEAD_DIM = 128
_N_BUCKETS = 256
_N_REP = 4
_DR = _N_REP * _HEAD_DIM


def _mm_nt_body(a_ref, b_ref, o_ref, *, out_dtype):
    o = lax.dot_general(
        a_ref[...].astype(jnp.bfloat16), b_ref[...].astype(jnp.bfloat16),
        (((1,), (1,)), ((), ())), preferred_element_type=jnp.float32)
    o_ref[...] = o.astype(out_dtype)


def _matmul_nt(a, b, block_n=512, out_dtype=jnp.float32):
    m, k = a.shape
    n, k2 = b.shape
    grid = (n // block_n,)
    return pl.pallas_call(
        functools.partial(_mm_nt_body, out_dtype=out_dtype),
        grid=grid,
        in_specs=[
            pl.BlockSpec((m, k), lambda j: (0, 0)),
            pl.BlockSpec((block_n, k), lambda j: (j, 0)),
        ],
        out_specs=pl.BlockSpec((m, block_n), lambda j: (0, j)),
        out_shape=jax.ShapeDtypeStruct((m, n), out_dtype),
    )(a, b)


def _mm_dual_body(a1_ref, a2_ref, w_ref, o_ref):
    w = w_ref[...].astype(jnp.bfloat16)
    o = lax.dot_general(
        a1_ref[...].astype(jnp.bfloat16), w[:, :_DR],
        (((1,), (1,)), ((), ())), preferred_element_type=jnp.float32)
    o += lax.dot_general(
        a2_ref[...].astype(jnp.bfloat16), w[:, _DR:],
        (((1,), (1,)), ((), ())), preferred_element_type=jnp.float32)
    o_ref[...] = o


def _matmul_dual(a1, a2, w, block_n=512):
    m = a1.shape[0]
    n, k = w.shape
    k2 = k - _DR
    grid = (n // block_n,)
    return pl.pallas_call(
        _mm_dual_body,
        grid=grid,
        in_specs=[
            pl.BlockSpec((m, _DR), lambda j: (0, 0)),
            pl.BlockSpec((m, k2), lambda j: (0, 0)),
            pl.BlockSpec((block_n, k), lambda j: (j, 0)),
        ],
        out_specs=pl.BlockSpec((m, block_n), lambda j: (0, j)),
        out_shape=jax.ShapeDtypeStruct((m, n), jnp.float32),
    )(a1, a2, w)


def _attn_body(q_ref, k_ref, v_ref, o_ref, *, scale):
    q = q_ref[...]
    k = k_ref[...]
    s = lax.dot_general(q, k, (((1,), (1,)), ((), ())),
                        preferred_element_type=jnp.float32) * scale
    m = jnp.max(s, axis=1, keepdims=True)
    p = jnp.exp(s - m)
    l = jnp.sum(p, axis=1, keepdims=True)
    o = lax.dot_general(p.astype(jnp.bfloat16),
                        v_ref[...].astype(jnp.bfloat16),
                        (((1,), (0,)), ((), ())),
                        preferred_element_type=jnp.float32)
    o_ref[...] = o / l


def _attention(q12, k12, v_all, block_q=256):
    s_len = q12.shape[0]
    d = _HEAD_DIM
    n_keep = _N_HEADS - _N_REP
    grid = (n_keep, s_len // block_q)
    return pl.pallas_call(
        functools.partial(_attn_body, scale=1.0 / math.sqrt(d)),
        grid=grid,
        in_specs=[
            pl.BlockSpec((block_q, d), lambda hh, qi: (qi, hh)),
            pl.BlockSpec((s_len, d), lambda hh, qi: (0, hh)),
            pl.BlockSpec((s_len, d), lambda hh, qi: (0, _N_REP + hh)),
        ],
        out_specs=pl.BlockSpec((block_q, d), lambda hh, qi: (qi, hh)),
        out_shape=jax.ShapeDtypeStruct((s_len, n_keep * d), jnp.float32),
    )(q12, k12, v_all)


def _sc_bucket(v_r, kh, qh):
    s_len = v_r.shape[0]
    n_tiles = 16
    rows_pt = s_len // n_tiles
    bk_pt = _N_BUCKETS // n_tiles
    sl = _DR // 128

    v3 = v_r.reshape(s_len, sl, 128)
    kh2 = kh.reshape(n_tiles, rows_pt)
    qh2 = qh.reshape(n_tiles, rows_pt)
    ones3 = jnp.ones((rows_pt, 1, 128), jnp.float32)

    mesh = plsc.VectorSubcoreMesh(core_axis_name="c", subcore_axis_name="s")

    @functools.partial(
        pl.kernel,
        out_type=(
            jax.ShapeDtypeStruct((s_len, sl, 128), jnp.float32),
            jax.ShapeDtypeStruct((_N_BUCKETS, sl, 128), jnp.float32),
        ),
        mesh=mesh,
        scratch_types=[
            pltpu.VMEM((rows_pt, sl, 128), jnp.float32),
            pltpu.VMEM((1, rows_pt), jnp.int32),
            pltpu.VMEM((1, rows_pt), jnp.int32),
            pltpu.VMEM((rows_pt, 1, 128), jnp.float32),
            pltpu.VMEM((bk_pt, sl, 128), jnp.float32),
            pltpu.VMEM((bk_pt, 1, 128), jnp.float32),
            pltpu.VMEM_SHARED((_N_BUCKETS, sl, 128), jnp.float32),
            pltpu.VMEM_SHARED((_N_BUCKETS, 1, 128), jnp.float32),
            pltpu.SemaphoreType.DMA,
        ],
    )
    def bucket_kernel(v_hbm, kh_hbm, qh_hbm, ones_hbm, out_hbm, pooled_hbm,
                      v_tile, kh_v, qh_v, ones_v, pool_t, cnt_t,
                      acc, acc_cnt, sem):
        cid = lax.axis_index("c")
        sid = lax.axis_index("s")

        @pl.when(cid == 0)
        def _():
            base = sid * rows_pt
            b0 = sid * bk_pt
            zero = jnp.zeros((16,), jnp.float32)

            for r in range(bk_pt):
                for c in range(0, 128, 16):
                    cnt_t[r, 0, pl.ds(c, 16)] = zero
                    for s in range(sl):
                        pool_t[r, s, pl.ds(c, 16)] = zero
            pltpu.sync_copy(pool_t, acc.at[pl.ds(b0, bk_pt)])
            pltpu.sync_copy(cnt_t, acc_cnt.at[pl.ds(b0, bk_pt)])

            pltpu.sync_copy(v_hbm.at[pl.ds(base, rows_pt)], v_tile)
            pltpu.sync_copy(kh_hbm.at[pl.ds(sid, 1)], kh_v)
            pltpu.sync_copy(qh_hbm.at[pl.ds(sid, 1)], qh_v)
            pltpu.sync_copy(ones_hbm, ones_v)

            plsc.subcore_barrier()

            pltpu.sync_copy(v_tile, acc.at[kh_v.at[0]], add=True)
            pltpu.sync_copy(ones_v, acc_cnt.at[kh_v.at[0]], add=True)

            plsc.subcore_barrier()

            pltpu.sync_copy(acc.at[pl.ds(b0, bk_pt)], pool_t)
            pltpu.sync_copy(acc_cnt.at[pl.ds(b0, bk_pt)], cnt_t)
            for r in range(bk_pt):
                rec = 1.0 / jnp.maximum(cnt_t[r, 0, pl.ds(0, 16)], 1.0)
                for s in range(sl):
                    for c in range(0, 128, 16):
                        pool_t[r, s, pl.ds(c, 16)] = (
                            pool_t[r, s, pl.ds(c, 16)] * rec)
            pltpu.sync_copy(pool_t, pooled_hbm.at[pl.ds(b0, bk_pt)])

            plsc.subcore_barrier()

            pltpu.async_copy(pooled_hbm.at[qh_v.at[0]], v_tile, sem).wait()
            pltpu.sync_copy(v_tile, out_hbm.at[pl.ds(base, rows_pt)])

    return bucket_kernel(v3, kh2, qh2, ones3)[0].reshape(s_len, _DR)


def kernel(hidden_states, Wq, Wk, Wv, Wo, key_horizons, query_horizons):
    b, s_len, h_dim = hidden_states.shape
    x = hidden_states.reshape(s_len, h_dim)

    q12 = _matmul_nt(x, Wq[_DR:], out_dtype=jnp.bfloat16)
    k12 = _matmul_nt(x, Wk[_DR:], out_dtype=jnp.bfloat16)
    v_all = _matmul_nt(x, Wv)

    bucket_out = _sc_bucket(v_all[:, :_DR], key_horizons, query_horizons)
    attn = _attention(q12, k12, v_all)

    out = _matmul_dual(bucket_out, attn, Wo)
    return out.reshape(b, s_len, h_dim)


# speedup vs baseline: 2.0876x; 1.0004x over previous
import functools
import math

import jax
import jax.numpy as jnp
from jax import lax
from jax.experimental import pallas as pl
from jax.experimental.pallas import tpu as pltpu
from jax.experimental.pallas import tpu_sc as plsc

_N_HEADS = 16
_HEAD_DIM = 128
_N_BUCKETS = 256
_N_REP = 4
_DR = _N_REP * _HEAD_DIM


def _mm_nt_body(a_ref, b_ref, o_ref, *, out_dtype):
    o = lax.dot_general(
        a_ref[...], b_ref[...].astype(jnp.bfloat16),
        (((1,), (1,)), ((), ())), preferred_element_type=jnp.float32)
    o_ref[...] = o.astype(out_dtype)


def _matmul_nt(a, b, block_n=512, out_dtype=jnp.float32):
    m, k = a.shape
    n, k2 = b.shape
    grid = (n // block_n,)
    return pl.pallas_call(
        functools.partial(_mm_nt_body, out_dtype=out_dtype),
        grid=grid,
        in_specs=[
            pl.BlockSpec((m, k), lambda j: (0, 0)),
            pl.BlockSpec((block_n, k), lambda j: (j, 0)),
        ],
        out_specs=pl.BlockSpec((m, block_n), lambda j: (0, j)),
        out_shape=jax.ShapeDtypeStruct((m, n), out_dtype),
    )(a, b)


def _mm_dual_body(a1_ref, a2_ref, w_ref, o_ref):
    w = w_ref[...].astype(jnp.bfloat16)
    o = lax.dot_general(
        a1_ref[...], w[:, :_DR],
        (((1,), (1,)), ((), ())), preferred_element_type=jnp.float32)
    o += lax.dot_general(
        a2_ref[...], w[:, _DR:],
        (((1,), (1,)), ((), ())), preferred_element_type=jnp.float32)
    o_ref[...] = o


def _matmul_dual(a1, a2, w, block_n=512):
    m = a1.shape[0]
    n, k = w.shape
    k2 = k - _DR
    grid = (n // block_n,)
    return pl.pallas_call(
        _mm_dual_body,
        grid=grid,
        in_specs=[
            pl.BlockSpec((m, _DR), lambda j: (0, 0)),
            pl.BlockSpec((m, k2), lambda j: (0, 0)),
            pl.BlockSpec((block_n, k), lambda j: (j, 0)),
        ],
        out_specs=pl.BlockSpec((m, block_n), lambda j: (0, j)),
        out_shape=jax.ShapeDtypeStruct((m, n), jnp.float32),
    )(a1, a2, w)


def _attn_body(q_ref, k_ref, v_ref, o_ref, *, scale):
    q = q_ref[...]
    k = k_ref[...]
    s = lax.dot_general(q, k, (((1,), (1,)), ((), ())),
                        preferred_element_type=jnp.float32) * scale
    m = jnp.max(s, axis=1, keepdims=True)
    p = jnp.exp(s - m)
    l = jnp.sum(p, axis=1, keepdims=True)
    o = lax.dot_general(p.astype(jnp.bfloat16),
                        v_ref[...].astype(jnp.bfloat16),
                        (((1,), (0,)), ((), ())),
                        preferred_element_type=jnp.float32)
    o_ref[...] = (o / l).astype(jnp.bfloat16)


def _attention(q12, k12, v_all, block_q=256):
    s_len = q12.shape[0]
    d = _HEAD_DIM
    n_keep = _N_HEADS - _N_REP
    grid = (n_keep, s_len // block_q)
    return pl.pallas_call(
        functools.partial(_attn_body, scale=1.0 / math.sqrt(d)),
        grid=grid,
        in_specs=[
            pl.BlockSpec((block_q, d), lambda hh, qi: (qi, hh)),
            pl.BlockSpec((s_len, d), lambda hh, qi: (0, hh)),
            pl.BlockSpec((s_len, d), lambda hh, qi: (0, _N_REP + hh)),
        ],
        out_specs=pl.BlockSpec((block_q, d), lambda hh, qi: (qi, hh)),
        out_shape=jax.ShapeDtypeStruct((s_len, n_keep * d), jnp.bfloat16),
    )(q12, k12, v_all)


def _sc_bucket(v_r, kh, qh):
    s_len = v_r.shape[0]
    n_tiles = 16
    rows_pt = s_len // n_tiles
    bk_pt = _N_BUCKETS // n_tiles
    sl = _DR // 128

    v3 = v_r.reshape(s_len, sl, 128)
    kh2 = kh.reshape(n_tiles, rows_pt)
    qh2 = qh.reshape(n_tiles, rows_pt)
    ones3 = jnp.ones((rows_pt, 1, 128), jnp.float32)

    mesh = plsc.VectorSubcoreMesh(core_axis_name="c", subcore_axis_name="s")

    @functools.partial(
        pl.kernel,
        out_type=(
            jax.ShapeDtypeStruct((s_len, sl, 128), jnp.float32),
            jax.ShapeDtypeStruct((_N_BUCKETS, sl, 128), jnp.float32),
        ),
        mesh=mesh,
        scratch_types=[
            pltpu.VMEM((rows_pt, sl, 128), jnp.float32),
            pltpu.VMEM((1, rows_pt), jnp.int32),
            pltpu.VMEM((1, rows_pt), jnp.int32),
            pltpu.VMEM((rows_pt, 1, 128), jnp.float32),
            pltpu.VMEM((bk_pt, sl, 128), jnp.float32),
            pltpu.VMEM((bk_pt, 1, 128), jnp.float32),
            pltpu.VMEM_SHARED((_N_BUCKETS, sl, 128), jnp.float32),
            pltpu.VMEM_SHARED((_N_BUCKETS, 1, 128), jnp.float32),
            pltpu.SemaphoreType.DMA,
        ],
    )
    def bucket_kernel(v_hbm, kh_hbm, qh_hbm, ones_hbm, out_hbm, pooled_hbm,
                      v_tile, kh_v, qh_v, ones_v, pool_t, cnt_t,
                      acc, acc_cnt, sem):
        cid = lax.axis_index("c")
        sid = lax.axis_index("s")

        @pl.when(cid == 0)
        def _():
            base = sid * rows_pt
            b0 = sid * bk_pt
            zero = jnp.zeros((16,), jnp.float32)

            for r in range(bk_pt):
                for c in range(0, 128, 16):
                    cnt_t[r, 0, pl.ds(c, 16)] = zero
                    for s in range(sl):
                        pool_t[r, s, pl.ds(c, 16)] = zero
            pltpu.sync_copy(pool_t, acc.at[pl.ds(b0, bk_pt)])
            pltpu.sync_copy(cnt_t, acc_cnt.at[pl.ds(b0, bk_pt)])

            pltpu.sync_copy(v_hbm.at[pl.ds(base, rows_pt)], v_tile)
            pltpu.sync_copy(kh_hbm.at[pl.ds(sid, 1)], kh_v)
            pltpu.sync_copy(qh_hbm.at[pl.ds(sid, 1)], qh_v)
            pltpu.sync_copy(ones_hbm, ones_v)

            plsc.subcore_barrier()

            pltpu.sync_copy(v_tile, acc.at[kh_v.at[0]], add=True)
            pltpu.sync_copy(ones_v, acc_cnt.at[kh_v.at[0]], add=True)

            plsc.subcore_barrier()

            pltpu.sync_copy(acc.at[pl.ds(b0, bk_pt)], pool_t)
            pltpu.sync_copy(acc_cnt.at[pl.ds(b0, bk_pt)], cnt_t)
            for r in range(bk_pt):
                rec = 1.0 / jnp.maximum(cnt_t[r, 0, pl.ds(0, 16)], 1.0)
                for s in range(sl):
                    for c in range(0, 128, 16):
                        pool_t[r, s, pl.ds(c, 16)] = (
                            pool_t[r, s, pl.ds(c, 16)] * rec)
            pltpu.sync_copy(pool_t, pooled_hbm.at[pl.ds(b0, bk_pt)])

            plsc.subcore_barrier()

            pltpu.async_copy(pooled_hbm.at[qh_v.at[0]], v_tile, sem).wait()
            pltpu.sync_copy(v_tile, out_hbm.at[pl.ds(base, rows_pt)])

    return bucket_kernel(v3, kh2, qh2, ones3)[0].reshape(s_len, _DR)


def kernel(hidden_states, Wq, Wk, Wv, Wo, key_horizons, query_horizons):
    b, s_len, h_dim = hidden_states.shape
    x = hidden_states.reshape(s_len, h_dim).astype(jnp.bfloat16)

    q12 = _matmul_nt(x, Wq[_DR:], out_dtype=jnp.bfloat16)
    k12 = _matmul_nt(x, Wk[_DR:], out_dtype=jnp.bfloat16)
    v_all = _matmul_nt(x, Wv)

    bucket_out = _sc_bucket(v_all[:, :_DR], key_horizons, query_horizons)
    attn = _attention(q12, k12, v_all)

    out = _matmul_dual(bucket_out.astype(jnp.bfloat16), attn, Wo)
    return out.reshape(b, s_len, h_dim)
